# Initial kernel scaffold; baseline (speedup 1.0000x reference)
import jax
import jax.numpy as jnp
from jax.experimental import pallas as pl


def kernel(x, Wr, br, W1, b1, W2, b2):
    raise NotImplementedError("write your pallas kernel here")


# speedup vs baseline: 2.6421x; 2.6421x over previous
import functools

import jax
import jax.numpy as jnp
from jax import lax
from jax.experimental import pallas as pl
from jax.experimental.pallas import tpu as pltpu
from jax.experimental.pallas import tpu_sc as plsc

B, S, H = 2, 2048, 1024
E, K = 16, 2
FF = 4 * H
T = B * S
CAP = 640
NSLOT = E * CAP
PACK_ROWS = (E + 1) * CAP
TRASH = NSLOT
TB = 512
NBLK = T // TB
FFB = 1024
NFB = FF // FFB

NC, NS = 2, 16
NW = NC * NS
TPW = T // NW
PCH = 32
NCH = TPW // PCH
TPW2 = T // (NW // 2)
GCH = 32
NGCH = TPW2 // GCH

_INV_SQRT2 = 0.7071067811865476


def _router_body(x_ref, wr_ref, br_ref, s0_ref, s1_ref, w0_ref, w1_ref, carry):
    i = pl.program_id(0)

    @pl.when(i == 0)
    def _():
        carry[...] = jnp.zeros_like(carry)

    x = x_ref[...].astype(jnp.bfloat16)
    logits = jnp.dot(x, wr_ref[...].astype(jnp.bfloat16),
                     preferred_element_type=jnp.float32) + br_ref[...]
    lane = lax.broadcasted_iota(jnp.int32, (TB, E), 1)
    i0 = jnp.argmax(logits, axis=1).astype(jnp.int32)
    v0 = jnp.max(logits, axis=1)
    masked = jnp.where(lane == i0[:, None], -jnp.inf, logits)
    i1 = jnp.argmax(masked, axis=1).astype(jnp.int32)
    v1 = jnp.max(masked, axis=1)
    g0 = 1.0 / (1.0 + jnp.exp(v1 - v0))
    g1 = 1.0 - g0
    oh0 = (lane == i0[:, None]).astype(jnp.float32)
    oh1 = (lane == i1[:, None]).astype(jnp.float32)
    ohs = oh0 + oh1
    r = lax.broadcasted_iota(jnp.int32, (TB, TB), 0)
    c = lax.broadcasted_iota(jnp.int32, (TB, TB), 1)
    tril = (c <= r).astype(jnp.float32)
    incl = jnp.dot(tril, ohs, preferred_element_type=jnp.float32,
                   precision=lax.Precision.HIGHEST)
    prev = carry[0:1, :]
    sexc = incl - ohs + prev
    carry[...] = jnp.broadcast_to(
        prev + jnp.sum(ohs, axis=0, keepdims=True), carry.shape)
    pos0 = jnp.sum(sexc * oh0, axis=1).astype(jnp.int32)
    pos1 = jnp.sum(sexc * oh1, axis=1).astype(jnp.int32)
    k0 = pos0 < CAP
    k1 = pos1 < CAP
    s0_ref[...] = jnp.where(k0, i0 * CAP + pos0, TRASH)[:, None]
    s1_ref[...] = jnp.where(k1, i1 * CAP + pos1, TRASH)[:, None]
    gk0 = jnp.where(k0, g0, 0.0)
    gk1 = jnp.where(k1, g1, 0.0)
    den = gk0 + gk1 + 1e-9
    w0_ref[...] = (gk0 / den)[:, None]
    w1_ref[...] = (gk1 / den)[:, None]


def _router(x2, wr, br2):
    return pl.pallas_call(
        _router_body,
        grid=(NBLK,),
        in_specs=[
            pl.BlockSpec((TB, H), lambda i: (i, 0)),
            pl.BlockSpec((H, E), lambda i: (0, 0)),
            pl.BlockSpec((1, E), lambda i: (0, 0)),
        ],
        out_specs=[
            pl.BlockSpec((TB, 1), lambda i: (i, 0)),
            pl.BlockSpec((TB, 1), lambda i: (i, 0)),
            pl.BlockSpec((TB, 1), lambda i: (i, 0)),
            pl.BlockSpec((TB, 1), lambda i: (i, 0)),
        ],
        out_shape=[
            jax.ShapeDtypeStruct((T, 1), jnp.int32),
            jax.ShapeDtypeStruct((T, 1), jnp.int32),
            jax.ShapeDtypeStruct((T, 1), jnp.float32),
            jax.ShapeDtypeStruct((T, 1), jnp.float32),
        ],
        scratch_shapes=[pltpu.VMEM((8, E), jnp.float32)],
    )(x2, wr, br2)


@functools.lru_cache(maxsize=None)
def _sc_kernels():
    mesh = plsc.VectorSubcoreMesh(core_axis_name="c", subcore_axis_name="s")

    @functools.partial(
        pl.kernel,
        mesh=mesh,
        out_type=jax.ShapeDtypeStruct((PACK_ROWS, H), jnp.float32),
        scratch_types=[
            pltpu.VMEM((PCH,), jnp.int32),
            pltpu.VMEM((PCH,), jnp.int32),
            pltpu.VMEM((PCH, H), jnp.float32),
            pltpu.SemaphoreType.DMA,
            pltpu.SemaphoreType.DMA,
        ],
    )
    def _sc_pack(x_hbm, slots_hbm, packed_hbm, s0_v, s1_v, rows_v, sem0, sem1):
        wid = lax.axis_index("s") * NC + lax.axis_index("c")
        base = wid * TPW
        for cidx in range(NCH):
            t0 = base + cidx * PCH
            pltpu.sync_copy(slots_hbm.at[pl.ds(t0, PCH)], s0_v)
            pltpu.sync_copy(slots_hbm.at[pl.ds(T + t0, PCH)], s1_v)
            pltpu.sync_copy(x_hbm.at[pl.ds(t0, PCH)], rows_v)
            c0 = pltpu.async_copy(rows_v, packed_hbm.at[s0_v], sem0)
            c1 = pltpu.async_copy(rows_v, packed_hbm.at[s1_v], sem1)
            c0.wait()
            c1.wait()

    @functools.partial(
        pl.kernel,
        mesh=mesh,
        out_type=jax.ShapeDtypeStruct((2 * T, H), jnp.float32),
        scratch_types=[
            pltpu.VMEM((GCH,), jnp.int32),
            pltpu.VMEM((GCH, H), jnp.float32),
            pltpu.SemaphoreType.DMA,
        ],
    )
    def _sc_gather(eflat_hbm, slots_hbm, y_hbm, idx_v, rows_v, sem):
        wid = lax.axis_index("s") * NC + lax.axis_index("c")
        plane = wid % 2
        base = (wid // 2) * TPW2
        for cidx in range(NGCH):
            t0 = base + cidx * GCH
            pltpu.sync_copy(slots_hbm.at[pl.ds(plane * T + t0, GCH)], idx_v)
            for i in range(GCH // 16):
                v = idx_v[pl.ds(i * 16, 16)]
                idx_v[pl.ds(i * 16, 16)] = jnp.minimum(v, NSLOT - 1)
            pltpu.async_copy(eflat_hbm.at[idx_v], rows_v, sem).wait()
            pltpu.sync_copy(rows_v, y_hbm.at[pl.ds(plane * T + t0, GCH)])

    return _sc_pack, _sc_gather


def _ffn1_body(x_ref, w1_ref, b1_ref, h_ref):
    xb = x_ref[...].astype(jnp.bfloat16)
    wb = w1_ref[0].astype(jnp.bfloat16)
    hm = jnp.dot(xb, wb, preferred_element_type=jnp.float32) + b1_ref[0]
    g = 0.5 * hm * (1.0 + lax.erf(hm * _INV_SQRT2))
    h_ref[...] = g.astype(jnp.bfloat16)


def _ffn1(packed, w1, b1):
    return pl.pallas_call(
        _ffn1_body,
        grid=(E, NFB),
        in_specs=[
            pl.BlockSpec((CAP, H), lambda e, f: (e, 0)),
            pl.BlockSpec((1, H, FFB), lambda e, f: (e, 0, f)),
            pl.BlockSpec((1, 1, FFB), lambda e, f: (e, 0, f)),
        ],
        out_specs=pl.BlockSpec((CAP, FFB), lambda e, f: (e, f)),
        out_shape=jax.ShapeDtypeStruct((NSLOT, FF), jnp.bfloat16),
    )(packed, w1, b1)


def _ffn2_body(h_ref, w2_ref, b2_ref, out_ref, acc):
    f = pl.program_id(1)

    @pl.when(f == 0)
    def _():
        acc[...] = jnp.zeros_like(acc)

    wb = w2_ref[0].astype(jnp.bfloat16)
    acc[...] += jnp.dot(h_ref[...], wb, preferred_element_type=jnp.float32)

    @pl.when(f == NFB - 1)
    def _():
        out_ref[...] = acc[...] + b2_ref[0]


def _ffn2(h, w2, b2):
    return pl.pallas_call(
        _ffn2_body,
        grid=(E, NFB),
        in_specs=[
            pl.BlockSpec((CAP, FFB), lambda e, f: (e, f)),
            pl.BlockSpec((1, FFB, H), lambda e, f: (e, f, 0)),
            pl.BlockSpec((1, 1, H), lambda e, f: (e, 0, 0)),
        ],
        out_specs=pl.BlockSpec((CAP, H), lambda e, f: (e, 0)),
        out_shape=jax.ShapeDtypeStruct((NSLOT, H), jnp.float32),
        scratch_shapes=[pltpu.VMEM((CAP, H), jnp.float32)],
    )(h, w2, b2)


def _combine_body(y0_ref, y1_ref, w0_ref, w1_ref, o_ref):
    w0 = w0_ref[...]
    w1 = w1_ref[...]
    a = jnp.where(w0 > 0.0, y0_ref[...] * w0, 0.0)
    b = jnp.where(w1 > 0.0, y1_ref[...] * w1, 0.0)
    o_ref[...] = a + b


def _combine(y, w0c, w1c):
    return pl.pallas_call(
        _combine_body,
        grid=(NBLK,),
        in_specs=[
            pl.BlockSpec((TB, H), lambda i: (i, 0)),
            pl.BlockSpec((TB, H), lambda i: (i + NBLK, 0)),
            pl.BlockSpec((TB, 1), lambda i: (i, 0)),
            pl.BlockSpec((TB, 1), lambda i: (i, 0)),
        ],
        out_specs=pl.BlockSpec((TB, H), lambda i: (i, 0)),
        out_shape=jax.ShapeDtypeStruct((T, H), jnp.float32),
    )(y, y, w0c, w1c)


def kernel(x, Wr, br, W1, b1, W2, b2):
    x2 = x.reshape(T, H)
    br2 = br.reshape(1, E)
    s0c, s1c, w0c, w1c = _router(x2, Wr, br2)
    slots = jnp.concatenate([s0c[:, 0], s1c[:, 0]], axis=0)
    sc_pack, sc_gather = _sc_kernels()
    packed = sc_pack(x2, slots)
    h = _ffn1(packed, W1, b1.reshape(E, 1, FF))
    eflat = _ffn2(h, W2, b2.reshape(E, 1, H))
    y = sc_gather(eflat, slots)
    out = _combine(y, w0c, w1c)
    return out.reshape(B, S, H)


# speedup vs baseline: 3.3494x; 1.2677x over previous
import functools

import jax
import jax.numpy as jnp
from jax import lax
from jax.experimental import pallas as pl
from jax.experimental.pallas import tpu as pltpu
from jax.experimental.pallas import tpu_sc as plsc

B, S, H = 2, 2048, 1024
E, K = 16, 2
FF = 4 * H
T = B * S
CAP = 640
NSLOT = E * CAP
PACK_ROWS = (E + 1) * CAP
TRASH = NSLOT
TB = 512
NBLK = T // TB
FFB = 1024
NFB = FF // FFB

NC, NS = 2, 16
NW = NC * NS
TPW = T // NW
PCH = 32
NCH = TPW // PCH
TPW2 = T // (NW // 2)
GCH = 32
NGCH = TPW2 // GCH

_INV_SQRT2 = 0.7071067811865476


def _router_body(x_ref, wr_ref, br_ref, s0_ref, s1_ref, w0_ref, w1_ref, carry):
    i = pl.program_id(0)

    @pl.when(i == 0)
    def _():
        carry[...] = jnp.zeros_like(carry)

    x = x_ref[...].astype(jnp.bfloat16)
    logits = jnp.dot(x, wr_ref[...].astype(jnp.bfloat16),
                     preferred_element_type=jnp.float32) + br_ref[...]
    lane = lax.broadcasted_iota(jnp.int32, (TB, E), 1)
    i0 = jnp.argmax(logits, axis=1).astype(jnp.int32)
    v0 = jnp.max(logits, axis=1)
    masked = jnp.where(lane == i0[:, None], -jnp.inf, logits)
    i1 = jnp.argmax(masked, axis=1).astype(jnp.int32)
    v1 = jnp.max(masked, axis=1)
    g0 = 1.0 / (1.0 + jnp.exp(v1 - v0))
    g1 = 1.0 - g0
    oh0 = (lane == i0[:, None]).astype(jnp.float32)
    oh1 = (lane == i1[:, None]).astype(jnp.float32)
    ohs = oh0 + oh1
    r = lax.broadcasted_iota(jnp.int32, (TB, TB), 0)
    c = lax.broadcasted_iota(jnp.int32, (TB, TB), 1)
    tril = (c <= r).astype(jnp.float32)
    incl = jnp.dot(tril, ohs, preferred_element_type=jnp.float32,
                   precision=lax.Precision.HIGHEST)
    prev = carry[0:1, :]
    sexc = incl - ohs + prev
    carry[...] = jnp.broadcast_to(
        prev + jnp.sum(ohs, axis=0, keepdims=True), carry.shape)
    pos0 = jnp.sum(sexc * oh0, axis=1).astype(jnp.int32)
    pos1 = jnp.sum(sexc * oh1, axis=1).astype(jnp.int32)
    k0 = pos0 < CAP
    k1 = pos1 < CAP
    s0_ref[...] = jnp.where(k0, i0 * CAP + pos0, TRASH)[:, None]
    s1_ref[...] = jnp.where(k1, i1 * CAP + pos1, TRASH)[:, None]
    gk0 = jnp.where(k0, g0, 0.0)
    gk1 = jnp.where(k1, g1, 0.0)
    den = gk0 + gk1 + 1e-9
    w0_ref[...] = (gk0 / den)[:, None]
    w1_ref[...] = (gk1 / den)[:, None]


def _router(x2, wr, br2):
    return pl.pallas_call(
        _router_body,
        grid=(NBLK,),
        in_specs=[
            pl.BlockSpec((TB, H), lambda i: (i, 0)),
            pl.BlockSpec((H, E), lambda i: (0, 0)),
            pl.BlockSpec((1, E), lambda i: (0, 0)),
        ],
        out_specs=[
            pl.BlockSpec((TB, 1), lambda i: (i, 0)),
            pl.BlockSpec((TB, 1), lambda i: (i, 0)),
            pl.BlockSpec((TB, 1), lambda i: (i, 0)),
            pl.BlockSpec((TB, 1), lambda i: (i, 0)),
        ],
        out_shape=[
            jax.ShapeDtypeStruct((T, 1), jnp.int32),
            jax.ShapeDtypeStruct((T, 1), jnp.int32),
            jax.ShapeDtypeStruct((T, 1), jnp.float32),
            jax.ShapeDtypeStruct((T, 1), jnp.float32),
        ],
        scratch_shapes=[pltpu.VMEM((8, E), jnp.float32)],
    )(x2, wr, br2)


@functools.lru_cache(maxsize=None)
def _sc_kernels():
    mesh = plsc.VectorSubcoreMesh(core_axis_name="c", subcore_axis_name="s")

    @functools.partial(
        pl.kernel,
        mesh=mesh,
        out_type=jax.ShapeDtypeStruct((PACK_ROWS, H), jnp.float32),
        scratch_types=[
            pltpu.VMEM((PCH,), jnp.int32),
            pltpu.VMEM((PCH,), jnp.int32),
            pltpu.VMEM((PCH, H), jnp.float32),
            pltpu.SemaphoreType.DMA,
            pltpu.SemaphoreType.DMA,
        ],
    )
    def _sc_pack(x_hbm, slots_hbm, packed_hbm, s0_v, s1_v, rows_v, sem0, sem1):
        wid = lax.axis_index("s") * NC + lax.axis_index("c")
        base = wid * TPW
        for cidx in range(NCH):
            t0 = base + cidx * PCH
            pltpu.sync_copy(slots_hbm.at[pl.ds(t0, PCH)], s0_v)
            pltpu.sync_copy(slots_hbm.at[pl.ds(T + t0, PCH)], s1_v)
            pltpu.sync_copy(x_hbm.at[pl.ds(t0, PCH)], rows_v)
            c0 = pltpu.async_copy(rows_v, packed_hbm.at[s0_v], sem0)
            c1 = pltpu.async_copy(rows_v, packed_hbm.at[s1_v], sem1)
            c0.wait()
            c1.wait()

    @functools.partial(
        pl.kernel,
        mesh=mesh,
        out_type=jax.ShapeDtypeStruct((2 * T, H), jnp.float32),
        scratch_types=[
            pltpu.VMEM((GCH,), jnp.int32),
            pltpu.VMEM((GCH, H), jnp.float32),
            pltpu.SemaphoreType.DMA,
        ],
    )
    def _sc_gather(eflat_hbm, slots_hbm, y_hbm, idx_v, rows_v, sem):
        wid = lax.axis_index("s") * NC + lax.axis_index("c")
        plane = wid % 2
        base = (wid // 2) * TPW2
        for cidx in range(NGCH):
            t0 = base + cidx * GCH
            pltpu.sync_copy(slots_hbm.at[pl.ds(plane * T + t0, GCH)], idx_v)
            for i in range(GCH // 16):
                v = idx_v[pl.ds(i * 16, 16)]
                idx_v[pl.ds(i * 16, 16)] = jnp.minimum(v, NSLOT - 1)
            pltpu.async_copy(eflat_hbm.at[idx_v], rows_v, sem).wait()
            pltpu.sync_copy(rows_v, y_hbm.at[pl.ds(plane * T + t0, GCH)])

    return _sc_pack, _sc_gather


def _ffn_body(x_ref, w1_ref, b1_ref, w2_ref, b2_ref, out_ref, acc):
    f = pl.program_id(1)

    @pl.when(f == 0)
    def _():
        acc[...] = jnp.zeros_like(acc)

    xb = x_ref[...].astype(jnp.bfloat16)
    hm = jnp.dot(xb, w1_ref[0].astype(jnp.bfloat16),
                 preferred_element_type=jnp.float32) + b1_ref[0]
    g = 0.5 * hm * (1.0 + lax.erf(hm * _INV_SQRT2))
    acc[...] += jnp.dot(g.astype(jnp.bfloat16), w2_ref[0].astype(jnp.bfloat16),
                        preferred_element_type=jnp.float32)

    @pl.when(f == NFB - 1)
    def _():
        out_ref[...] = acc[...] + b2_ref[0]


def _ffn(packed, w1, b1, w2, b2):
    return pl.pallas_call(
        _ffn_body,
        grid=(E, NFB),
        in_specs=[
            pl.BlockSpec((CAP, H), lambda e, f: (e, 0)),
            pl.BlockSpec((1, H, FFB), lambda e, f: (e, 0, f)),
            pl.BlockSpec((1, 1, FFB), lambda e, f: (e, 0, f)),
            pl.BlockSpec((1, FFB, H), lambda e, f: (e, f, 0)),
            pl.BlockSpec((1, 1, H), lambda e, f: (e, 0, 0)),
        ],
        out_specs=pl.BlockSpec((CAP, H), lambda e, f: (e, 0)),
        out_shape=jax.ShapeDtypeStruct((NSLOT, H), jnp.float32),
        scratch_shapes=[pltpu.VMEM((CAP, H), jnp.float32)],
    )(packed, w1, b1, w2, b2)


def _combine_body(y0_ref, y1_ref, w0_ref, w1_ref, o_ref):
    w0 = w0_ref[...]
    w1 = w1_ref[...]
    a = jnp.where(w0 > 0.0, y0_ref[...] * w0, 0.0)
    b = jnp.where(w1 > 0.0, y1_ref[...] * w1, 0.0)
    o_ref[...] = a + b


def _combine(y, w0c, w1c):
    return pl.pallas_call(
        _combine_body,
        grid=(NBLK,),
        in_specs=[
            pl.BlockSpec((TB, H), lambda i: (i, 0)),
            pl.BlockSpec((TB, H), lambda i: (i + NBLK, 0)),
            pl.BlockSpec((TB, 1), lambda i: (i, 0)),
            pl.BlockSpec((TB, 1), lambda i: (i, 0)),
        ],
        out_specs=pl.BlockSpec((TB, H), lambda i: (i, 0)),
        out_shape=jax.ShapeDtypeStruct((T, H), jnp.float32),
    )(y, y, w0c, w1c)


def kernel(x, Wr, br, W1, b1, W2, b2):
    x2 = x.reshape(T, H)
    br2 = br.reshape(1, E)
    s0c, s1c, w0c, w1c = _router(x2, Wr, br2)
    slots = jnp.concatenate([s0c[:, 0], s1c[:, 0]], axis=0)
    sc_pack, sc_gather = _sc_kernels()
    packed = sc_pack(x2, slots)
    eflat = _ffn(packed, W1, b1.reshape(E, 1, FF), W2, b2.reshape(E, 1, H))
    y = sc_gather(eflat, slots)
    out = _combine(y, w0c, w1c)
    return out.reshape(B, S, H)


# speedup vs baseline: 3.5290x; 1.0536x over previous
import functools

import jax
import jax.numpy as jnp
from jax import lax
from jax.experimental import pallas as pl
from jax.experimental.pallas import tpu as pltpu
from jax.experimental.pallas import tpu_sc as plsc

B, S, H = 2, 2048, 1024
E, K = 16, 2
FF = 4 * H
T = B * S
CAP = 640
NSLOT = E * CAP
PACK_ROWS = (E + 1) * CAP
TRASH = NSLOT
TB = 512
NBLK = T // TB
FFB = 2048
NFB = FF // FFB

NC, NS = 2, 16
NW = NC * NS
TPW = T // NW
PCH = 32
NCH = TPW // PCH
TPW2 = T // (NW // 2)
GCH = 32
NGCH = TPW2 // GCH

_INV_SQRT2 = 0.7071067811865476


def _router_body(x_ref, wr_ref, br_ref, s0_ref, s1_ref, w0_ref, w1_ref, carry):
    i = pl.program_id(0)

    @pl.when(i == 0)
    def _():
        carry[...] = jnp.zeros_like(carry)

    x = x_ref[...].astype(jnp.bfloat16)
    logits = jnp.dot(x, wr_ref[...].astype(jnp.bfloat16),
                     preferred_element_type=jnp.float32) + br_ref[...]
    lane = lax.broadcasted_iota(jnp.int32, (TB, E), 1)
    i0 = jnp.argmax(logits, axis=1).astype(jnp.int32)
    v0 = jnp.max(logits, axis=1)
    masked = jnp.where(lane == i0[:, None], -jnp.inf, logits)
    i1 = jnp.argmax(masked, axis=1).astype(jnp.int32)
    v1 = jnp.max(masked, axis=1)
    g0 = 1.0 / (1.0 + jnp.exp(v1 - v0))
    g1 = 1.0 - g0
    oh0 = (lane == i0[:, None]).astype(jnp.float32)
    oh1 = (lane == i1[:, None]).astype(jnp.float32)
    ohs = oh0 + oh1
    r = lax.broadcasted_iota(jnp.int32, (TB, TB), 0)
    c = lax.broadcasted_iota(jnp.int32, (TB, TB), 1)
    tril = (c <= r).astype(jnp.float32)
    incl = jnp.dot(tril, ohs, preferred_element_type=jnp.float32,
                   precision=lax.Precision.HIGHEST)
    prev = carry[0:1, :]
    sexc = incl - ohs + prev
    carry[...] = jnp.broadcast_to(
        prev + jnp.sum(ohs, axis=0, keepdims=True), carry.shape)
    pos0 = jnp.sum(sexc * oh0, axis=1).astype(jnp.int32)
    pos1 = jnp.sum(sexc * oh1, axis=1).astype(jnp.int32)
    k0 = pos0 < CAP
    k1 = pos1 < CAP
    s0_ref[...] = jnp.where(k0, i0 * CAP + pos0, TRASH)[:, None]
    s1_ref[...] = jnp.where(k1, i1 * CAP + pos1, TRASH)[:, None]
    gk0 = jnp.where(k0, g0, 0.0)
    gk1 = jnp.where(k1, g1, 0.0)
    den = gk0 + gk1 + 1e-9
    w0_ref[...] = (gk0 / den)[:, None]
    w1_ref[...] = (gk1 / den)[:, None]


def _router(x2, wr, br2):
    return pl.pallas_call(
        _router_body,
        grid=(NBLK,),
        in_specs=[
            pl.BlockSpec((TB, H), lambda i: (i, 0)),
            pl.BlockSpec((H, E), lambda i: (0, 0)),
            pl.BlockSpec((1, E), lambda i: (0, 0)),
        ],
        out_specs=[
            pl.BlockSpec((TB, 1), lambda i: (i, 0)),
            pl.BlockSpec((TB, 1), lambda i: (i, 0)),
            pl.BlockSpec((TB, 1), lambda i: (i, 0)),
            pl.BlockSpec((TB, 1), lambda i: (i, 0)),
        ],
        out_shape=[
            jax.ShapeDtypeStruct((T, 1), jnp.int32),
            jax.ShapeDtypeStruct((T, 1), jnp.int32),
            jax.ShapeDtypeStruct((T, 1), jnp.float32),
            jax.ShapeDtypeStruct((T, 1), jnp.float32),
        ],
        scratch_shapes=[pltpu.VMEM((8, E), jnp.float32)],
    )(x2, wr, br2)


@functools.lru_cache(maxsize=None)
def _sc_kernels():
    mesh = plsc.VectorSubcoreMesh(core_axis_name="c", subcore_axis_name="s")

    @functools.partial(
        pl.kernel,
        mesh=mesh,
        out_type=jax.ShapeDtypeStruct((PACK_ROWS, H), jnp.float32),
        scratch_types=[
            pltpu.VMEM((PCH,), jnp.int32),
            pltpu.VMEM((PCH,), jnp.int32),
            pltpu.VMEM((PCH, H), jnp.float32),
            pltpu.SemaphoreType.DMA,
            pltpu.SemaphoreType.DMA,
        ],
    )
    def _sc_pack(x_hbm, slots_hbm, packed_hbm, s0_v, s1_v, rows_v, sem0, sem1):
        wid = lax.axis_index("s") * NC + lax.axis_index("c")
        base = wid * TPW
        for cidx in range(NCH):
            t0 = base + cidx * PCH
            pltpu.sync_copy(slots_hbm.at[pl.ds(t0, PCH)], s0_v)
            pltpu.sync_copy(slots_hbm.at[pl.ds(T + t0, PCH)], s1_v)
            pltpu.sync_copy(x_hbm.at[pl.ds(t0, PCH)], rows_v)
            c0 = pltpu.async_copy(rows_v, packed_hbm.at[s0_v], sem0)
            c1 = pltpu.async_copy(rows_v, packed_hbm.at[s1_v], sem1)
            c0.wait()
            c1.wait()

    @functools.partial(
        pl.kernel,
        mesh=mesh,
        out_type=jax.ShapeDtypeStruct((2 * T, H), jnp.float32),
        scratch_types=[
            pltpu.VMEM((GCH,), jnp.int32),
            pltpu.VMEM((GCH, H), jnp.float32),
            pltpu.SemaphoreType.DMA,
        ],
    )
    def _sc_gather(eflat_hbm, slots_hbm, y_hbm, idx_v, rows_v, sem):
        wid = lax.axis_index("s") * NC + lax.axis_index("c")
        plane = wid % 2
        base = (wid // 2) * TPW2
        for cidx in range(NGCH):
            t0 = base + cidx * GCH
            pltpu.sync_copy(slots_hbm.at[pl.ds(plane * T + t0, GCH)], idx_v)
            for i in range(GCH // 16):
                v = idx_v[pl.ds(i * 16, 16)]
                idx_v[pl.ds(i * 16, 16)] = jnp.minimum(v, NSLOT - 1)
            pltpu.async_copy(eflat_hbm.at[idx_v], rows_v, sem).wait()
            pltpu.sync_copy(rows_v, y_hbm.at[pl.ds(plane * T + t0, GCH)])

    return _sc_pack, _sc_gather


def _ffn_body(x_ref, w1_ref, b1_ref, w2_ref, b2_ref, out_ref, acc):
    f = pl.program_id(1)

    @pl.when(f == 0)
    def _():
        acc[...] = jnp.zeros_like(acc)

    xb = x_ref[...].astype(jnp.bfloat16)
    hm = jnp.dot(xb, w1_ref[0].astype(jnp.bfloat16),
                 preferred_element_type=jnp.float32) + b1_ref[0]
    g = 0.5 * hm * (1.0 + lax.erf(hm * _INV_SQRT2))
    acc[...] += jnp.dot(g.astype(jnp.bfloat16), w2_ref[0].astype(jnp.bfloat16),
                        preferred_element_type=jnp.float32)

    @pl.when(f == NFB - 1)
    def _():
        out_ref[...] = acc[...] + b2_ref[0]


def _ffn(packed, w1, b1, w2, b2):
    return pl.pallas_call(
        _ffn_body,
        grid=(E, NFB),
        in_specs=[
            pl.BlockSpec((CAP, H), lambda e, f: (e, 0)),
            pl.BlockSpec((1, H, FFB), lambda e, f: (e, 0, f)),
            pl.BlockSpec((1, 1, FFB), lambda e, f: (e, 0, f)),
            pl.BlockSpec((1, FFB, H), lambda e, f: (e, f, 0)),
            pl.BlockSpec((1, 1, H), lambda e, f: (e, 0, 0)),
        ],
        out_specs=pl.BlockSpec((CAP, H), lambda e, f: (e, 0)),
        out_shape=jax.ShapeDtypeStruct((NSLOT, H), jnp.float32),
        scratch_shapes=[pltpu.VMEM((CAP, H), jnp.float32)],
    )(packed, w1, b1, w2, b2)


def _combine_body(y0_ref, y1_ref, w0_ref, w1_ref, o_ref):
    w0 = w0_ref[...]
    w1 = w1_ref[...]
    a = jnp.where(w0 > 0.0, y0_ref[...] * w0, 0.0)
    b = jnp.where(w1 > 0.0, y1_ref[...] * w1, 0.0)
    o_ref[...] = a + b


def _combine(y, w0c, w1c):
    return pl.pallas_call(
        _combine_body,
        grid=(NBLK,),
        in_specs=[
            pl.BlockSpec((TB, H), lambda i: (i, 0)),
            pl.BlockSpec((TB, H), lambda i: (i + NBLK, 0)),
            pl.BlockSpec((TB, 1), lambda i: (i, 0)),
            pl.BlockSpec((TB, 1), lambda i: (i, 0)),
        ],
        out_specs=pl.BlockSpec((TB, H), lambda i: (i, 0)),
        out_shape=jax.ShapeDtypeStruct((T, H), jnp.float32),
    )(y, y, w0c, w1c)


def kernel(x, Wr, br, W1, b1, W2, b2):
    x2 = x.reshape(T, H)
    br2 = br.reshape(1, E)
    s0c, s1c, w0c, w1c = _router(x2, Wr, br2)
    slots = jnp.concatenate([s0c[:, 0], s1c[:, 0]], axis=0)
    sc_pack, sc_gather = _sc_kernels()
    packed = sc_pack(x2, slots)
    eflat = _ffn(packed, W1, b1.reshape(E, 1, FF), W2, b2.reshape(E, 1, H))
    y = sc_gather(eflat, slots)
    out = _combine(y, w0c, w1c)
    return out.reshape(B, S, H)


# speedup vs baseline: 3.5415x; 1.0035x over previous
import functools

import jax
import jax.numpy as jnp
from jax import lax
from jax.experimental import pallas as pl
from jax.experimental.pallas import tpu as pltpu
from jax.experimental.pallas import tpu_sc as plsc

B, S, H = 2, 2048, 1024
E, K = 16, 2
FF = 4 * H
T = B * S
CAP = 640
NSLOT = E * CAP
PACK_ROWS = (E + 1) * CAP
TRASH = NSLOT
TB = 512
NBLK = T // TB
FFB = 2048
NFB = FF // FFB

NC, NS = 2, 16
NW = NC * NS
TPW = T // NW
PCH = 32
NCH = TPW // PCH
TPW2 = T // (NW // 2)
GCH = 32
NGCH = TPW2 // GCH

_INV_SQRT2 = 0.7071067811865476


def _router_body(x_ref, wr_ref, br_ref, s0_ref, s1_ref, w0_ref, w1_ref, carry):
    i = pl.program_id(0)

    @pl.when(i == 0)
    def _():
        carry[...] = jnp.zeros_like(carry)

    x = x_ref[...].astype(jnp.bfloat16)
    logits = jnp.dot(x, wr_ref[...].astype(jnp.bfloat16),
                     preferred_element_type=jnp.float32) + br_ref[...]
    lane = lax.broadcasted_iota(jnp.int32, (TB, E), 1)
    i0 = jnp.argmax(logits, axis=1).astype(jnp.int32)
    v0 = jnp.max(logits, axis=1)
    masked = jnp.where(lane == i0[:, None], -jnp.inf, logits)
    i1 = jnp.argmax(masked, axis=1).astype(jnp.int32)
    v1 = jnp.max(masked, axis=1)
    g0 = 1.0 / (1.0 + jnp.exp(v1 - v0))
    g1 = 1.0 - g0
    oh0 = (lane == i0[:, None]).astype(jnp.float32)
    oh1 = (lane == i1[:, None]).astype(jnp.float32)
    ohs = oh0 + oh1
    r = lax.broadcasted_iota(jnp.int32, (TB, TB), 0)
    c = lax.broadcasted_iota(jnp.int32, (TB, TB), 1)
    tril = (c <= r).astype(jnp.float32)
    incl = jnp.dot(tril, ohs, preferred_element_type=jnp.float32,
                   precision=lax.Precision.HIGHEST)
    prev = carry[0:1, :]
    sexc = incl - ohs + prev
    carry[...] = jnp.broadcast_to(
        prev + jnp.sum(ohs, axis=0, keepdims=True), carry.shape)
    pos0 = jnp.sum(sexc * oh0, axis=1).astype(jnp.int32)
    pos1 = jnp.sum(sexc * oh1, axis=1).astype(jnp.int32)
    k0 = pos0 < CAP
    k1 = pos1 < CAP
    s0_ref[...] = jnp.where(k0, i0 * CAP + pos0, TRASH)[:, None]
    s1_ref[...] = jnp.where(k1, i1 * CAP + pos1, TRASH)[:, None]
    gk0 = jnp.where(k0, g0, 0.0)
    gk1 = jnp.where(k1, g1, 0.0)
    den = gk0 + gk1 + 1e-9
    w0_ref[...] = (gk0 / den)[:, None]
    w1_ref[...] = (gk1 / den)[:, None]


def _router(x2, wr, br2):
    return pl.pallas_call(
        _router_body,
        grid=(NBLK,),
        in_specs=[
            pl.BlockSpec((TB, H), lambda i: (i, 0)),
            pl.BlockSpec((H, E), lambda i: (0, 0)),
            pl.BlockSpec((1, E), lambda i: (0, 0)),
        ],
        out_specs=[
            pl.BlockSpec((TB, 1), lambda i: (i, 0)),
            pl.BlockSpec((TB, 1), lambda i: (i, 0)),
            pl.BlockSpec((TB, 1), lambda i: (i, 0)),
            pl.BlockSpec((TB, 1), lambda i: (i, 0)),
        ],
        out_shape=[
            jax.ShapeDtypeStruct((T, 1), jnp.int32),
            jax.ShapeDtypeStruct((T, 1), jnp.int32),
            jax.ShapeDtypeStruct((T, 1), jnp.float32),
            jax.ShapeDtypeStruct((T, 1), jnp.float32),
        ],
        scratch_shapes=[pltpu.VMEM((8, E), jnp.float32)],
    )(x2, wr, br2)


@functools.lru_cache(maxsize=None)
def _sc_kernels():
    mesh = plsc.VectorSubcoreMesh(core_axis_name="c", subcore_axis_name="s")

    @functools.partial(
        pl.kernel,
        mesh=mesh,
        out_type=jax.ShapeDtypeStruct((PACK_ROWS, H), jnp.float32),
        scratch_types=[
            pltpu.VMEM((2, PCH), jnp.int32),
            pltpu.VMEM((2, PCH), jnp.int32),
            pltpu.VMEM((2, PCH, H), jnp.float32),
            pltpu.SemaphoreType.DMA,
            pltpu.SemaphoreType.DMA,
        ],
    )
    def _sc_pack(x_hbm, s0_hbm, s1_hbm, packed_hbm, s0_v, s1_v, rows_v,
                 sem0, sem1):
        wid = lax.axis_index("s") * NC + lax.axis_index("c")
        base = wid * TPW

        def load(cidx, buf):
            t0 = base + cidx * PCH
            pltpu.sync_copy(s0_hbm.at[pl.ds(t0, PCH)], s0_v.at[buf])
            pltpu.sync_copy(s1_hbm.at[pl.ds(t0, PCH)], s1_v.at[buf])
            pltpu.sync_copy(x_hbm.at[pl.ds(t0, PCH)], rows_v.at[buf])

        load(0, 0)
        for cidx in range(NCH):
            buf = cidx % 2
            c0 = pltpu.async_copy(rows_v.at[buf], packed_hbm.at[s0_v.at[buf]],
                                  sem0)
            c1 = pltpu.async_copy(rows_v.at[buf], packed_hbm.at[s1_v.at[buf]],
                                  sem1)
            if cidx + 1 < NCH:
                load(cidx + 1, 1 - buf)
            c0.wait()
            c1.wait()

    @functools.partial(
        pl.kernel,
        mesh=mesh,
        out_type=jax.ShapeDtypeStruct((2 * T, H), jnp.float32),
        scratch_types=[
            pltpu.VMEM((2, GCH), jnp.int32),
            pltpu.VMEM((2, GCH, H), jnp.float32),
            pltpu.SemaphoreType.DMA,
            pltpu.SemaphoreType.DMA,
        ],
    )
    def _sc_gather(eflat_hbm, s0_hbm, s1_hbm, y_hbm, idx_v, rows_v,
                   sem0, sem1):
        wid = lax.axis_index("s") * NC + lax.axis_index("c")
        base = wid * TPW
        sems = (sem0, sem1)

        def start(j, buf):
            plane, cidx = j % 2, j // 2
            t0 = base + cidx * PCH
            src = s0_hbm if plane == 0 else s1_hbm
            pltpu.sync_copy(src.at[pl.ds(t0, PCH)], idx_v.at[buf])
            for i in range(PCH // 16):
                v = idx_v[buf, pl.ds(i * 16, 16)]
                idx_v[buf, pl.ds(i * 16, 16)] = jnp.minimum(v, NSLOT - 1)
            return pltpu.async_copy(eflat_hbm.at[idx_v.at[buf]],
                                    rows_v.at[buf], sems[buf])

        njob = 2 * NCH
        cp = start(0, 0)
        for j in range(njob):
            buf = j % 2
            cp.wait()
            nxt = start(j + 1, 1 - buf) if j + 1 < njob else None
            plane, cidx = j % 2, j // 2
            t0 = base + cidx * PCH
            pltpu.sync_copy(rows_v.at[buf], y_hbm.at[pl.ds(plane * T + t0,
                                                           PCH)])
            cp = nxt

    return _sc_pack, _sc_gather


def _ffn_body(x_ref, w1_ref, b1_ref, w2_ref, b2_ref, out_ref, acc):
    f = pl.program_id(1)

    @pl.when(f == 0)
    def _():
        acc[...] = jnp.zeros_like(acc)

    xb = x_ref[...].astype(jnp.bfloat16)
    hm = jnp.dot(xb, w1_ref[0].astype(jnp.bfloat16),
                 preferred_element_type=jnp.float32) + b1_ref[0]
    g = 0.5 * hm * (1.0 + lax.erf(hm * _INV_SQRT2))
    acc[...] += jnp.dot(g.astype(jnp.bfloat16), w2_ref[0].astype(jnp.bfloat16),
                        preferred_element_type=jnp.float32)

    @pl.when(f == NFB - 1)
    def _():
        out_ref[...] = acc[...] + b2_ref[0]


def _ffn(packed, w1, b1, w2, b2):
    return pl.pallas_call(
        _ffn_body,
        grid=(E, NFB),
        in_specs=[
            pl.BlockSpec((CAP, H), lambda e, f: (e, 0)),
            pl.BlockSpec((1, H, FFB), lambda e, f: (e, 0, f)),
            pl.BlockSpec((1, 1, FFB), lambda e, f: (e, 0, f)),
            pl.BlockSpec((1, FFB, H), lambda e, f: (e, f, 0)),
            pl.BlockSpec((1, 1, H), lambda e, f: (e, 0, 0)),
        ],
        out_specs=pl.BlockSpec((CAP, H), lambda e, f: (e, 0)),
        out_shape=jax.ShapeDtypeStruct((NSLOT, H), jnp.float32),
        scratch_shapes=[pltpu.VMEM((CAP, H), jnp.float32)],
    )(packed, w1, b1, w2, b2)


def _combine_body(y0_ref, y1_ref, w0_ref, w1_ref, o_ref):
    w0 = w0_ref[...]
    w1 = w1_ref[...]
    a = jnp.where(w0 > 0.0, y0_ref[...] * w0, 0.0)
    b = jnp.where(w1 > 0.0, y1_ref[...] * w1, 0.0)
    o_ref[...] = a + b


def _combine(y, w0c, w1c):
    return pl.pallas_call(
        _combine_body,
        grid=(NBLK,),
        in_specs=[
            pl.BlockSpec((TB, H), lambda i: (i, 0)),
            pl.BlockSpec((TB, H), lambda i: (i + NBLK, 0)),
            pl.BlockSpec((TB, 1), lambda i: (i, 0)),
            pl.BlockSpec((TB, 1), lambda i: (i, 0)),
        ],
        out_specs=pl.BlockSpec((TB, H), lambda i: (i, 0)),
        out_shape=jax.ShapeDtypeStruct((T, H), jnp.float32),
    )(y, y, w0c, w1c)


def kernel(x, Wr, br, W1, b1, W2, b2):
    x2 = x.reshape(T, H)
    br2 = br.reshape(1, E)
    s0c, s1c, w0c, w1c = _router(x2, Wr, br2)
    s0, s1 = s0c.reshape(T), s1c.reshape(T)
    sc_pack, sc_gather = _sc_kernels()
    packed = sc_pack(x2, s0, s1)
    eflat = _ffn(packed, W1, b1.reshape(E, 1, FF), W2, b2.reshape(E, 1, H))
    y = sc_gather(eflat, s0, s1)
    out = _combine(y, w0c, w1c)
    return out.reshape(B, S, H)
